# baseline (speedup 1.0000x reference)
import jax
import jax.numpy as jnp
from jax import lax
from jax.experimental import pallas as pl
from jax.experimental.pallas import tpu as pltpu
from jax.experimental.pallas import tpu_sc as plsc

BATCH = 16384
D = 32
OUT_D = D + 1
NS = 16
L = 16
PER_TILE = BATCH // NS
NCHUNK = 8
CHUNK = PER_TILE // NCHUNK
FLAT = PER_TILE * OUT_D


def _process(ids_hbm, table_hbm, out_hbm, s, idx_v, rows_v, out_v, sem):
    pltpu.sync_copy(ids_hbm.at[s], idx_v)
    copies = []
    for k in range(NCHUNK):
        copies.append(
            pltpu.async_copy(
                table_hbm.at[idx_v.at[k]],
                rows_v.at[pl.ds(k * CHUNK, CHUNK)],
                sem,
            )
        )
    col_step = lax.iota(jnp.int32, L) * OUT_D
    for k in range(NCHUNK):
        for j in range(CHUNK // L):
            ids = idx_v[k, pl.ds(j * L, L)]
            pos = (k * CHUNK + j * L) * OUT_D + col_step
            plsc.store_scatter(out_v, [pos], ids.astype(jnp.float32))
    for c in copies:
        c.wait()

    def row_body(i, _):
        base = i * OUT_D
        out_v[pl.ds(base + 1, L)] = rows_v[i, pl.ds(0, L)]
        out_v[pl.ds(base + 1 + L, L)] = rows_v[i, pl.ds(L, L)]
        return 0

    lax.fori_loop(0, PER_TILE, row_body, 0, unroll=4)
    pltpu.sync_copy(out_v, out_hbm.at[pl.ds(s * FLAT, FLAT)])


def _body(uid_hbm, iid_hbm, users_hbm, items_hbm, out_u_hbm, out_i_hbm,
          idx_v, rows_v, out_v, sem):
    c = lax.axis_index("c")
    s = lax.axis_index("s")

    @pl.when(c == 0)
    def _():
        _process(uid_hbm, users_hbm, out_u_hbm, s, idx_v, rows_v, out_v, sem)

    @pl.when(c == 1)
    def _():
        _process(iid_hbm, items_hbm, out_i_hbm, s, idx_v, rows_v, out_v, sem)


@jax.jit
def _sc_lookup(uid, iid, users, items):
    mesh = plsc.VectorSubcoreMesh(core_axis_name="c", subcore_axis_name="s")
    f = pl.kernel(
        _body,
        out_type=(
            jax.ShapeDtypeStruct((BATCH * OUT_D,), jnp.float32),
            jax.ShapeDtypeStruct((BATCH * OUT_D,), jnp.float32),
        ),
        mesh=mesh,
        compiler_params=pltpu.CompilerParams(
            needs_layout_passes=False, use_tc_tiling_on_sc=False
        ),
        scratch_types=[
            pltpu.VMEM((NCHUNK, CHUNK), jnp.int32),
            pltpu.VMEM((PER_TILE, D), jnp.float32),
            pltpu.VMEM((FLAT,), jnp.float32),
            pltpu.SemaphoreType.DMA,
        ],
    )
    return f(uid, iid, users, items)


def kernel(user_id, item_id, users, items):
    uid = user_id.reshape(NS, NCHUNK, CHUNK)
    iid = item_id.reshape(NS, NCHUNK, CHUNK)
    out_u, out_i = _sc_lookup(uid, iid, users, items)
    return (out_u.reshape(BATCH, OUT_D), out_i.reshape(BATCH, OUT_D))
